# Initial kernel scaffold; baseline (speedup 1.0000x reference)
import jax
import jax.numpy as jnp
from jax.experimental import pallas as pl


def kernel(x, channel_indices):
    raise NotImplementedError("write your pallas kernel here")


# speedup vs baseline: 1.0037x; 1.0037x over previous
import functools

import jax
import jax.numpy as jnp
from jax import lax
from jax.experimental import pallas as pl
from jax.experimental.pallas import tpu as pltpu
from jax.experimental.pallas import tpu_sc as plsc

NC = 2
NS = 16
NW = NC * NS

SPLIT = 8
GROUP = 8
ACTIVE = 31
GROUPS_PER_W = 28
CHUNK_D = 6272
N_CHUNKS = 6944


def _body(x_hbm, widx_hbm, out_hbm, idx_v, buf0, buf1, gsem, wsem0, wsem1):
    w = lax.axis_index("s") * NC + lax.axis_index("c")

    @pl.when(w < ACTIVE)
    def _():
        pltpu.sync_copy(widx_hbm.at[w], idx_v)
        bufs = (buf0, buf1)
        wsems = (wsem0, wsem1)
        base = w * (GROUPS_PER_W * GROUP)

        def do_group(g, slot, first):
            if not first:
                pltpu.make_async_copy(
                    bufs[slot], out_hbm.at[pl.ds(0, GROUP)], wsems[slot]
                ).wait()
            pltpu.async_copy(x_hbm.at[idx_v.at[g]], bufs[slot], gsem).wait()
            row = pl.multiple_of(base + g * GROUP, GROUP)
            pltpu.async_copy(bufs[slot], out_hbm.at[pl.ds(row, GROUP)], wsems[slot])

        do_group(jnp.int32(0), 0, True)
        do_group(jnp.int32(1), 1, True)

        def outer(o, carry):
            do_group(o * 2, 0, False)
            do_group(o * 2 + 1, 1, False)
            return carry

        lax.fori_loop(1, GROUPS_PER_W // 2, outer, jnp.int32(0))

        pltpu.make_async_copy(buf0, out_hbm.at[pl.ds(0, GROUP)], wsem0).wait()
        pltpu.make_async_copy(buf1, out_hbm.at[pl.ds(0, GROUP)], wsem1).wait()


_sc_gather = functools.partial(
    pl.kernel,
    out_type=jax.ShapeDtypeStruct((N_CHUNKS, CHUNK_D), jnp.float32),
    mesh=plsc.VectorSubcoreMesh(
        core_axis_name="c", subcore_axis_name="s", num_cores=NC, num_subcores=NS
    ),
    scratch_types=[
        pltpu.VMEM((GROUPS_PER_W, GROUP), jnp.int32),
        pltpu.VMEM((GROUP, CHUNK_D), jnp.float32),
        pltpu.VMEM((GROUP, CHUNK_D), jnp.float32),
        pltpu.SemaphoreType.DMA,
        pltpu.SemaphoreType.DMA,
        pltpu.SemaphoreType.DMA,
    ],
)(_body)


def kernel(x, channel_indices):
    B, C, H, W = x.shape
    T = channel_indices.shape[1]
    assert (B, C, H, W) == (4, 31, 224, 224) and T == 7
    D = H * W // SPLIT

    x2 = x.reshape(B * C * SPLIT, D)
    flat = channel_indices.reshape(-1).astype(jnp.int32)
    src_rows = (jnp.arange(B, dtype=jnp.int32)[:, None] * C
                + flat[None, :]).reshape(-1)
    src_chunk = (src_rows[:, None] * SPLIT
                 + jnp.arange(SPLIT, dtype=jnp.int32)[None, :]).reshape(-1)
    pad = NW * GROUPS_PER_W * GROUP - N_CHUNKS
    widx = jnp.concatenate([src_chunk, jnp.zeros((pad,), jnp.int32)])
    widx = widx.reshape(NW, GROUPS_PER_W, GROUP)

    out2 = _sc_gather(x2, widx)
    return out2.reshape(B, C, T, H, W)


# speedup vs baseline: 2.4880x; 2.4789x over previous
import functools

import jax
import jax.numpy as jnp
from jax import lax
from jax.experimental import pallas as pl
from jax.experimental.pallas import tpu as pltpu
from jax.experimental.pallas import tpu_sc as plsc

NC = 2
NS = 16
NW = NC * NS

ACTIVE = 31
PLANES_PER_W = 28
B, C, H, W = 4, 31, 224, 224
T = 7
N_PLANES = B * C * T
TR = H // 8


def _body(x_hbm, widx_hbm, out_hbm, idx_v, buf0, buf1, gsem, wsem0, wsem1):
    w = lax.axis_index("s") * NC + lax.axis_index("c")

    @pl.when(w < ACTIVE)
    def _():
        pltpu.sync_copy(widx_hbm.at[w], idx_v)
        bufs = (buf0, buf1)
        wsems = (wsem0, wsem1)
        base = w * PLANES_PER_W

        def do_plane(g, slot):
            @pl.when(g >= 2)
            def _drain():
                pltpu.make_async_copy(
                    bufs[slot], out_hbm.at[pl.ds(0, TR)], wsems[slot]
                ).wait()

            lanes = plsc.load_gather(idx_v, [jnp.broadcast_to(g, (16,))])
            p = jnp.max(lanes)
            src = pl.multiple_of(p * TR, TR)
            pltpu.async_copy(x_hbm.at[pl.ds(src, TR)], bufs[slot], gsem).wait()
            row = pl.multiple_of((base + g) * TR, TR)
            pltpu.async_copy(bufs[slot], out_hbm.at[pl.ds(row, TR)], wsems[slot])

        def outer(o, carry):
            do_plane(o * 2, 0)
            do_plane(o * 2 + 1, 1)
            return carry

        lax.fori_loop(0, PLANES_PER_W // 2, outer, jnp.int32(0))

        pltpu.make_async_copy(buf0, out_hbm.at[pl.ds(0, 1)], wsem0).wait()
        pltpu.make_async_copy(buf1, out_hbm.at[pl.ds(0, 1)], wsem1).wait()


_sc_gather = functools.partial(
    pl.kernel,
    out_type=jax.ShapeDtypeStruct((N_PLANES * TR, 8, W), jnp.float32),
    mesh=plsc.VectorSubcoreMesh(
        core_axis_name="c", subcore_axis_name="s", num_cores=NC, num_subcores=NS
    ),
    scratch_types=[
        pltpu.VMEM((PLANES_PER_W,), jnp.int32),
        pltpu.VMEM((TR, 8, W), jnp.float32),
        pltpu.VMEM((TR, 8, W), jnp.float32),
        pltpu.SemaphoreType.DMA,
        pltpu.SemaphoreType.DMA,
        pltpu.SemaphoreType.DMA,
    ],
    compiler_params=pltpu.CompilerParams(
        use_tc_tiling_on_sc=True, needs_layout_passes=False
    ),
)(_body)


def kernel(x, channel_indices):
    assert x.shape == (B, C, H, W) and channel_indices.shape == (C, T)

    x3 = x.reshape(B * C * TR, 8, W)
    flat = channel_indices.reshape(-1).astype(jnp.int32)
    src_planes = (jnp.arange(B, dtype=jnp.int32)[:, None] * C
                  + flat[None, :]).reshape(-1)
    pad = NW * PLANES_PER_W - N_PLANES
    widx = jnp.concatenate([src_planes, jnp.zeros((pad,), jnp.int32)])
    widx = widx.reshape(NW, PLANES_PER_W)

    out3 = _sc_gather(x3, widx)
    return out3.reshape(B, C, T, H, W)
